# baseline (speedup 1.0000x reference)
import functools

import jax
import jax.numpy as jnp
from jax import lax
from jax.experimental import pallas as pl
from jax.experimental.pallas import tpu as pltpu
from jax.experimental.pallas import tpu_sc as plsc

NC = 2
NS = 16
NW = NC * NS

CHUNK = 128
K = 8
S = 4


def _make_gather(n, vocab, dim):
    assert n % (NW * CHUNK * K) == 0
    per_w = n // NW
    n_chunks = per_w // CHUNK
    n_groups = n_chunks // K

    mesh = plsc.VectorSubcoreMesh(core_axis_name="c", subcore_axis_name="s")

    @functools.partial(
        pl.kernel,
        out_type=jax.ShapeDtypeStruct((n, dim), jnp.float32),
        mesh=mesh,
        compiler_params=pltpu.CompilerParams(use_tc_tiling_on_sc=False),
        scratch_types=[
            pltpu.VMEM((per_w,), jnp.int32),
            pltpu.VMEM((K, CHUNK, dim), jnp.float32),
        ] + [pltpu.SemaphoreType.DMA] * (2 * K),
    )
    def gather_kernel(idx_hbm, table_hbm, out_hbm, idx_v, rows_v, *sems):
        gsem, wsem = sems[:K], sems[K:]
        wid = lax.axis_index("s") * NC + lax.axis_index("c")
        base = wid * per_w

        pltpu.sync_copy(idx_hbm.at[pl.ds(base, per_w)], idx_v)

        def fire_gather(c, b):
            pltpu.async_copy(
                table_hbm.at[idx_v.at[pl.ds(c * CHUNK, CHUNK)]],
                rows_v.at[b], gsem[b])

        def wait_gather(c, b):
            pltpu.make_async_copy(
                table_hbm.at[idx_v.at[pl.ds(c * CHUNK, CHUNK)]],
                rows_v.at[b], gsem[b]).wait()

        def fire_write(c, b):
            pltpu.async_copy(
                rows_v.at[b], out_hbm.at[pl.ds(base + c * CHUNK, CHUNK)],
                wsem[b])

        def wait_write(c, b):
            pltpu.make_async_copy(
                rows_v.at[b], out_hbm.at[pl.ds(base + c * CHUNK, CHUNK)],
                wsem[b]).wait()

        for b in range(K):
            fire_gather(b, b)

        def group_body(g, _):
            for b in range(K):
                j = (g - 1) * K + b
                wait_gather(j, b)
                fire_write(j, b)
            for b in range(K):
                j = (g - 1) * K + b
                wait_write(j, b)
                fire_gather(g * K + b, b)
            return 0

        lax.fori_loop(1, n_groups, group_body, 0)

        for b in range(K):
            j = (n_groups - 1) * K + b
            wait_gather(j, b)
            fire_write(j, b)
        for b in range(K):
            j = (n_groups - 1) * K + b
            wait_write(j, b)

    return gather_kernel


def kernel(x, table):
    vocab, dim = table.shape
    if x.ndim != 4:
        raise ValueError(f"TokenEmbedding expects 4D input [B, H, W, C]. Got: {x.shape}")
    B, H, W = x.shape[0], x.shape[1], x.shape[2]
    if x.shape[-1] == vocab:
        xi = jnp.argmax(x, axis=-1).astype(jnp.int32).reshape(-1)
        n = B * H * W
    else:
        xi = x.astype(jnp.int32).reshape(-1)
        n = B * H * W * x.shape[3]
    ns = n // S
    gather = _make_gather(ns, vocab, dim)
    outs = [gather(xi[i * ns:(i + 1) * ns], table) for i in range(S)]
    return jnp.concatenate(outs, axis=0).reshape(B, H, W, dim)


# speedup vs baseline: 1.4266x; 1.4266x over previous
import functools

import jax
import jax.numpy as jnp
from jax import lax
from jax.experimental import pallas as pl
from jax.experimental.pallas import tpu as pltpu
from jax.experimental.pallas import tpu_sc as plsc

NC = 2
NS = 16
NW = NC * NS

CHUNK = 128
K = 8


def _make_gather(n, vocab, dim):
    assert n % (NW * CHUNK * K) == 0
    per_w = n // NW
    n_chunks = per_w // CHUNK
    n_groups = n_chunks // K

    mesh = plsc.VectorSubcoreMesh(core_axis_name="c", subcore_axis_name="s")

    @functools.partial(
        pl.kernel,
        out_type=jax.ShapeDtypeStruct((n, dim), jnp.float32),
        mesh=mesh,
        compiler_params=pltpu.CompilerParams(use_tc_tiling_on_sc=False),
        scratch_types=[
            pltpu.VMEM((per_w,), jnp.int32),
            pltpu.VMEM((K, CHUNK, dim), jnp.float32),
        ] + [pltpu.SemaphoreType.DMA] * (2 * K),
    )
    def gather_kernel(idx_hbm, table_hbm, out_hbm, idx_v, rows_v, *sems):
        gsem, wsem = sems[:K], sems[K:]
        wid = lax.axis_index("s") * NC + lax.axis_index("c")
        base = wid * per_w

        pltpu.sync_copy(idx_hbm.at[pl.ds(base, per_w)], idx_v)

        def fire_gather(c, b):
            pltpu.async_copy(
                table_hbm.at[idx_v.at[pl.ds(c * CHUNK, CHUNK)]],
                rows_v.at[b], gsem[b])

        def wait_gather(c, b):
            pltpu.make_async_copy(
                table_hbm.at[idx_v.at[pl.ds(c * CHUNK, CHUNK)]],
                rows_v.at[b], gsem[b]).wait()

        def fire_write(c, b):
            pltpu.async_copy(
                rows_v.at[b], out_hbm.at[pl.ds(base + c * CHUNK, CHUNK)],
                wsem[b])

        def wait_write(c, b):
            pltpu.make_async_copy(
                rows_v.at[b], out_hbm.at[pl.ds(base + c * CHUNK, CHUNK)],
                wsem[b]).wait()

        for b in range(K):
            fire_gather(b, b)

        def group_body(g, _):
            for b in range(K):
                j = (g - 1) * K + b
                wait_gather(j, b)
                fire_write(j, b)
            for b in range(K):
                j = (g - 1) * K + b
                wait_write(j, b)
                fire_gather(g * K + b, b)
            return 0

        lax.fori_loop(1, n_groups, group_body, 0)

        for b in range(K):
            j = (n_groups - 1) * K + b
            wait_gather(j, b)
            fire_write(j, b)
        for b in range(K):
            j = (n_groups - 1) * K + b
            wait_write(j, b)

    return gather_kernel


def kernel(x, table):
    vocab, dim = table.shape
    if x.ndim != 4:
        raise ValueError(f"TokenEmbedding expects 4D input [B, H, W, C]. Got: {x.shape}")
    B, H, W = x.shape[0], x.shape[1], x.shape[2]
    if x.shape[-1] == vocab:
        xi = jnp.argmax(x, axis=-1).astype(jnp.int32).reshape(-1)
        n = B * H * W
    else:
        xi = x.astype(jnp.int32).reshape(-1)
        n = B * H * W * x.shape[3]
    out = _make_gather(n, vocab, dim)(xi, table)
    return out.reshape(B, H, W, dim)
